# baseline (speedup 1.0000x reference)
import functools

import jax
import jax.numpy as jnp
import numpy as np
from jax import lax
from jax.experimental import pallas as pl
from jax.experimental.pallas import tpu as pltpu
from jax.experimental.pallas import tpu_sc as plsc

_N = 20000
_ROWS = 160
_LANES = 128
_PAD = _ROWS * _LANES
_K = 1000
_SCORE_THRESH = 0.05
_NMS_THRESH = 0.5
_DETS = 300
_OUT_ROWS = 304

_NEG = -1.0
_BIGI = 2**30
_NEGF = -3.0e38

_NTILES = 16
_PER_TILE = _PAD // _NTILES
_IDXROWS = _PER_TILE // _LANES
_CAP = 1024
_OUTSZ = _CAP + _PAD


def _select_kernel(s_ref, sel_ref, oidx_ref):
    shape = (_ROWS, _LANES)
    row_id = lax.broadcasted_iota(jnp.int32, shape, 0)
    lane_id = lax.broadcasted_iota(jnp.int32, shape, 1)
    idx = row_id * _LANES + lane_id
    valid = idx < _N

    probs = jax.nn.sigmoid(s_ref[:])
    probs = jnp.where(probs >= _SCORE_THRESH, probs, _NEG)
    probs = jnp.where(valid, probs, _NEG)

    bits = lax.bitcast_convert_type(probs, jnp.int32)
    keys = jnp.where(probs >= 0.0, bits, np.int32(-1))
    keys = jnp.where(valid, keys, np.int32(-2))

    def bs_body(_, lh):
        lo, hi = lh
        mid = lo + (hi - lo) // 2
        c = jnp.sum(jnp.where(keys > mid, 1, 0))
        take_hi = c < _K
        return (jnp.where(take_hi, lo, mid), jnp.where(take_hi, mid, hi))

    _, t_key = lax.fori_loop(0, 31, bs_body, (np.int32(-2), np.int32(2**30)))

    c_gt = jnp.sum(jnp.where(keys > t_key, 1, 0))
    k_rem = _K - c_gt
    is_tie = keys == t_key

    def ts_body(_, lh):
        lo, hi = lh
        mid = lo + (hi - lo) // 2
        c = jnp.sum(jnp.where(is_tie & (idx < mid), 1, 0))
        take_hi = c >= k_rem
        return (jnp.where(take_hi, lo, mid), jnp.where(take_hi, mid, hi))

    _, m_hi = lax.fori_loop(0, 16, ts_body, (np.int32(0), np.int32(_PAD)))
    m_idx = jnp.where(k_rem > 0, m_hi, np.int32(0))

    cand = (keys > t_key) | (is_tie & (idx < m_idx))
    sel = jnp.where(cand, probs, _NEG)
    sel_ref[:] = sel

    live = (sel > 0.0).astype(jnp.float32)
    tri_incl = (lax.broadcasted_iota(jnp.int32, (_LANES, _LANES), 0)
                <= lax.broadcasted_iota(jnp.int32, (_LANES, _LANES), 1)
                ).astype(jnp.float32)
    within = jax.lax.dot_general(
        live, tri_incl, (((1,), (0,)), ((), ())),
        preferred_element_type=jnp.float32)
    rowtot = within[:, _LANES - 1:_LANES]
    tri_lt = (lax.broadcasted_iota(jnp.int32, (_ROWS, _ROWS), 1)
              < lax.broadcasted_iota(jnp.int32, (_ROWS, _ROWS), 0)
              ).astype(jnp.float32)
    rowpre = jax.lax.dot_general(
        tri_lt, rowtot, (((1,), (0,)), ((), ())),
        preferred_element_type=jnp.float32)
    pos = (within + rowpre - 1.0).astype(jnp.int32)
    oidx_ref[:] = jnp.where(sel > 0.0, pos, _CAP + idx)


def _shuffle_body(sel_hbm, x1_hbm, y1_hbm, x2_hbm, y2_hbm, oidx_hbm,
                  osel_hbm, ox1_hbm, oy1_hbm, ox2_hbm, oy2_hbm,
                  sel_v, c1_v, c2_v, c3_v, c4_v, idx2d, initbuf,
                  sh0, sh1, sh2, sh3, sh4, sem):
    wid = lax.axis_index("s")
    base = wid * _PER_TILE

    pltpu.sync_copy(oidx_hbm.at[wid], idx2d)
    pltpu.sync_copy(sel_hbm.at[pl.ds(base, _PER_TILE)], sel_v)
    pltpu.sync_copy(x1_hbm.at[pl.ds(base, _PER_TILE)], c1_v)
    pltpu.sync_copy(y1_hbm.at[pl.ds(base, _PER_TILE)], c2_v)
    pltpu.sync_copy(x2_hbm.at[pl.ds(base, _PER_TILE)], c3_v)
    pltpu.sync_copy(y2_hbm.at[pl.ds(base, _PER_TILE)], c4_v)

    @pl.when(wid == 0)
    def _():
        fill = jnp.full((16,), _NEG, jnp.float32)
        for u in range(_CAP // 16):
            initbuf[pl.ds(u * 16, 16)] = fill
        pltpu.sync_copy(initbuf, sh0.at[pl.ds(0, _CAP)])
        pltpu.sync_copy(initbuf, sh1.at[pl.ds(0, _CAP)])
        pltpu.sync_copy(initbuf, sh2.at[pl.ds(0, _CAP)])
        pltpu.sync_copy(initbuf, sh3.at[pl.ds(0, _CAP)])
        pltpu.sync_copy(initbuf, sh4.at[pl.ds(0, _CAP)])

    plsc.subcore_barrier()

    copies = []
    for j in range(_IDXROWS):
        s = pl.ds(j * _LANES, _LANES)
        for vbuf, shb in ((sel_v, sh0), (c1_v, sh1), (c2_v, sh2),
                          (c3_v, sh3), (c4_v, sh4)):
            copies.append(
                pltpu.async_copy(vbuf.at[s], shb.at[idx2d.at[j]], sem))
    for c in copies:
        c.wait()

    plsc.subcore_barrier()

    for a, (shb, ohbm) in enumerate(
            ((sh0, osel_hbm), (sh1, ox1_hbm), (sh2, oy1_hbm),
             (sh3, ox2_hbm), (sh4, oy2_hbm))):
        @pl.when(wid == a)
        def _(shb=shb, ohbm=ohbm):
            pltpu.sync_copy(shb.at[pl.ds(0, _CAP)], ohbm)


@functools.lru_cache(maxsize=1)
def _get_sc_shuffle():
    mesh = plsc.VectorSubcoreMesh(
        core_axis_name="c", subcore_axis_name="s",
        num_cores=1, num_subcores=_NTILES)
    return pl.kernel(
        _shuffle_body,
        out_type=tuple(
            jax.ShapeDtypeStruct((_CAP,), jnp.float32) for _ in range(5)),
        mesh=mesh,
        scratch_types=[
            pltpu.VMEM((_PER_TILE,), jnp.float32),
            pltpu.VMEM((_PER_TILE,), jnp.float32),
            pltpu.VMEM((_PER_TILE,), jnp.float32),
            pltpu.VMEM((_PER_TILE,), jnp.float32),
            pltpu.VMEM((_PER_TILE,), jnp.float32),
            pltpu.VMEM((_IDXROWS, _LANES), jnp.int32),
            pltpu.VMEM((_CAP,), jnp.float32),
            pltpu.VMEM_SHARED((_OUTSZ,), jnp.float32),
            pltpu.VMEM_SHARED((_OUTSZ,), jnp.float32),
            pltpu.VMEM_SHARED((_OUTSZ,), jnp.float32),
            pltpu.VMEM_SHARED((_OUTSZ,), jnp.float32),
            pltpu.VMEM_SHARED((_OUTSZ,), jnp.float32),
            pltpu.SemaphoreType.DMA,
        ],
    )


def _nms1024_kernel(sel_ref, x1_ref, y1_ref, x2_ref, y2_ref, out_ref):
    shape = (_CAP // _LANES, _LANES)
    row_id = lax.broadcasted_iota(jnp.int32, shape, 0)
    lane_id = lax.broadcasted_iota(jnp.int32, shape, 1)
    idx = row_id * _LANES + lane_id

    x1 = x1_ref[:]
    y1 = y1_ref[:]
    x2 = x2_ref[:]
    y2 = y2_ref[:]
    a2 = jnp.maximum(x2 - x1, 0.0) * jnp.maximum(y2 - y1, 0.0)

    def nms_body(t, sel):
        m = jnp.max(sel, keepdims=True)
        j = jnp.min(jnp.where(sel == m, idx, np.int32(_BIGI)), keepdims=True)
        is_j = idx == j
        bx1 = jnp.max(jnp.where(is_j, x1, _NEGF), keepdims=True)
        by1 = jnp.max(jnp.where(is_j, y1, _NEGF), keepdims=True)
        bx2 = jnp.max(jnp.where(is_j, x2, _NEGF), keepdims=True)
        by2 = jnp.max(jnp.where(is_j, y2, _NEGF), keepdims=True)

        out_lane = lax.broadcasted_iota(jnp.int32, (1, _LANES), 1)
        vals = jnp.where(out_lane == 0, bx1, 0.0)
        vals = jnp.where(out_lane == 1, by1, vals)
        vals = jnp.where(out_lane == 2, bx2, vals)
        vals = jnp.where(out_lane == 3, by2, vals)
        vals = jnp.where(out_lane == 4, m, vals)
        vals = jnp.where(m > 0.0, vals, 0.0)
        out_ref[pl.ds(t, 1), :] = vals

        xx1 = jnp.maximum(bx1, x1)
        yy1 = jnp.maximum(by1, y1)
        xx2 = jnp.minimum(bx2, x2)
        yy2 = jnp.minimum(by2, y2)
        inter = jnp.maximum(xx2 - xx1, 0.0) * jnp.maximum(yy2 - yy1, 0.0)
        a1 = jnp.maximum(bx2 - bx1, 0.0) * jnp.maximum(by2 - by1, 0.0)
        union = jnp.maximum(a1 + a2 - inter, 1e-8)
        iou = inter / union
        supp = (iou > _NMS_THRESH) | is_j
        return jnp.where(supp, _NEG, sel)

    lax.fori_loop(0, _DETS, nms_body, sel_ref[:])


@functools.partial(jax.jit, static_argnames=())
def kernel(boxes, scores):
    pad = _PAD - _N
    planes = [jnp.pad(boxes[:, c], (0, pad)) for c in range(4)]
    s2d = jnp.pad(scores, (0, pad)).reshape(_ROWS, _LANES)

    sel2d, oidx = pl.pallas_call(
        _select_kernel,
        out_shape=(jax.ShapeDtypeStruct((_ROWS, _LANES), jnp.float32),
                   jax.ShapeDtypeStruct((_ROWS, _LANES), jnp.int32)),
    )(s2d)

    co = _get_sc_shuffle()(sel2d.reshape(_PAD), *planes,
                           oidx.reshape(_NTILES, _IDXROWS, _LANES))
    osel, ox1, oy1, ox2, oy2 = [
        a.reshape(_CAP // _LANES, _LANES) for a in co]

    out = pl.pallas_call(
        _nms1024_kernel,
        out_shape=jax.ShapeDtypeStruct((_OUT_ROWS, _LANES), jnp.float32),
    )(osel, ox1, oy1, ox2, oy2)

    out_boxes = out[:_DETS, :4]
    out_scores = out[:_DETS, 4]
    return out_boxes, out_scores


# speedup vs baseline: 2.1352x; 2.1352x over previous
import functools

import jax
import jax.numpy as jnp
import numpy as np
from jax import lax
from jax.experimental import pallas as pl
from jax.experimental.pallas import tpu as pltpu
from jax.experimental.pallas import tpu_sc as plsc

_N = 20000
_ROWS = 160
_LANES = 128
_PAD = _ROWS * _LANES
_K = 1000
_SCORE_THRESH = 0.05
_NMS_THRESH = 0.5
_DETS = 300
_OUT_ROWS = 304

_NEG = -1.0
_BIGI = 2**30
_NEGF = -3.0e38

_NTILES = 16
_PER_TILE = _PAD // _NTILES
_IDXROWS = _PER_TILE // _LANES
_CAP = 1024
_OUTSZ = _CAP + _PAD


_TBL = _CAP * _LANES
_TBLSZ = _TBL + _PAD


def _select_kernel(s_ref, sel_ref, oidx_ref, oidxb_ref):
    shape = (_ROWS, _LANES)
    row_id = lax.broadcasted_iota(jnp.int32, shape, 0)
    lane_id = lax.broadcasted_iota(jnp.int32, shape, 1)
    idx = row_id * _LANES + lane_id
    valid = idx < _N

    probs = jax.nn.sigmoid(s_ref[:])
    probs = jnp.where(probs >= _SCORE_THRESH, probs, _NEG)
    probs = jnp.where(valid, probs, _NEG)

    bits = lax.bitcast_convert_type(probs, jnp.int32)
    keys = jnp.where(probs >= 0.0, bits, np.int32(-1))
    keys = jnp.where(valid, keys, np.int32(-2))

    def bs_body(_, lh):
        lo, hi = lh
        mid = lo + (hi - lo) // 2
        c = jnp.sum(jnp.where(keys > mid, 1, 0))
        take_hi = c < _K
        return (jnp.where(take_hi, lo, mid), jnp.where(take_hi, mid, hi))

    _, t_key = lax.fori_loop(0, 31, bs_body, (np.int32(-2), np.int32(2**30)))

    c_gt = jnp.sum(jnp.where(keys > t_key, 1, 0))
    k_rem = _K - c_gt
    is_tie = keys == t_key

    def ts_body(_, lh):
        lo, hi = lh
        mid = lo + (hi - lo) // 2
        c = jnp.sum(jnp.where(is_tie & (idx < mid), 1, 0))
        take_hi = c >= k_rem
        return (jnp.where(take_hi, lo, mid), jnp.where(take_hi, mid, hi))

    _, m_hi = lax.fori_loop(0, 16, ts_body, (np.int32(0), np.int32(_PAD)))
    m_idx = jnp.where(k_rem > 0, m_hi, np.int32(0))

    cand = (keys > t_key) | (is_tie & (idx < m_idx))
    sel = jnp.where(cand, probs, _NEG)
    sel_ref[:] = sel

    live = (sel > 0.0).astype(jnp.float32)
    tri_incl = (lax.broadcasted_iota(jnp.int32, (_LANES, _LANES), 0)
                <= lax.broadcasted_iota(jnp.int32, (_LANES, _LANES), 1)
                ).astype(jnp.float32)
    within = jax.lax.dot_general(
        live, tri_incl, (((1,), (0,)), ((), ())),
        preferred_element_type=jnp.float32)
    rowtot = within[:, _LANES - 1:_LANES]
    tri_lt = (lax.broadcasted_iota(jnp.int32, (_ROWS, _ROWS), 1)
              < lax.broadcasted_iota(jnp.int32, (_ROWS, _ROWS), 0)
              ).astype(jnp.float32)
    rowpre = jax.lax.dot_general(
        tri_lt, rowtot, (((1,), (0,)), ((), ())),
        preferred_element_type=jnp.float32)
    pos = (within + rowpre - 1.0).astype(jnp.int32)
    oidx_ref[:] = jnp.where(sel > 0.0, pos, _CAP + idx)
    oidxb_ref[:] = jnp.where(sel > 0.0, pos * _LANES, _TBL + idx)


def _shuffle_body(sel_hbm, x1_hbm, y1_hbm, x2_hbm, y2_hbm, oidx_hbm,
                  oidxb_hbm,
                  osel_hbm, ox1_hbm, oy1_hbm, ox2_hbm, oy2_hbm, tbl_hbm,
                  sel_v, c1_v, c2_v, c3_v, c4_v, idx2d, idxb2d, idxbc,
                  initbuf, initz,
                  sh0, sh1, sh2, sh3, sh4, tbl_sh, sem):
    wid = lax.axis_index("s")
    base = wid * _PER_TILE

    pltpu.sync_copy(oidx_hbm.at[wid], idx2d)
    pltpu.sync_copy(oidxb_hbm.at[wid], idxb2d)
    pltpu.sync_copy(sel_hbm.at[pl.ds(base, _PER_TILE)], sel_v)
    pltpu.sync_copy(x1_hbm.at[pl.ds(base, _PER_TILE)], c1_v)
    pltpu.sync_copy(y1_hbm.at[pl.ds(base, _PER_TILE)], c2_v)
    pltpu.sync_copy(x2_hbm.at[pl.ds(base, _PER_TILE)], c3_v)
    pltpu.sync_copy(y2_hbm.at[pl.ds(base, _PER_TILE)], c4_v)

    for j in range(_IDXROWS):
        for u in range(_LANES // 16):
            v = idxb2d[j, pl.ds(u * 16, 16)]
            for c in range(5):
                idxbc[c, j, pl.ds(u * 16, 16)] = v + c

    @pl.when(wid == 0)
    def _():
        fill = jnp.full((16,), _NEG, jnp.float32)
        for u in range(_CAP // 16):
            initbuf[pl.ds(u * 16, 16)] = fill
        pltpu.sync_copy(initbuf, sh0.at[pl.ds(0, _CAP)])
        pltpu.sync_copy(initbuf, sh1.at[pl.ds(0, _CAP)])
        pltpu.sync_copy(initbuf, sh2.at[pl.ds(0, _CAP)])
        pltpu.sync_copy(initbuf, sh3.at[pl.ds(0, _CAP)])
        pltpu.sync_copy(initbuf, sh4.at[pl.ds(0, _CAP)])

    zfill = jnp.zeros((16,), jnp.float32)
    for u in range(_CAP // 16):
        initz[pl.ds(u * 16, 16)] = zfill
    seg = _TBL // _NTILES
    for k in range(seg // _CAP):
        pltpu.sync_copy(initz, tbl_sh.at[pl.ds(wid * seg + k * _CAP, _CAP)])

    plsc.subcore_barrier()

    copies = []
    for j in range(_IDXROWS):
        s = pl.ds(j * _LANES, _LANES)
        for vbuf, shb in ((sel_v, sh0), (c1_v, sh1), (c2_v, sh2),
                          (c3_v, sh3), (c4_v, sh4)):
            copies.append(
                pltpu.async_copy(vbuf.at[s], shb.at[idx2d.at[j]], sem))
        for c, vbuf in enumerate((c1_v, c2_v, c3_v, c4_v, sel_v)):
            copies.append(
                pltpu.async_copy(vbuf.at[s], tbl_sh.at[idxbc.at[c, j]], sem))
    for c in copies:
        c.wait()

    plsc.subcore_barrier()

    for a, (shb, ohbm) in enumerate(
            ((sh0, osel_hbm), (sh1, ox1_hbm), (sh2, oy1_hbm),
             (sh3, ox2_hbm), (sh4, oy2_hbm))):
        @pl.when(wid == a)
        def _(shb=shb, ohbm=ohbm):
            pltpu.sync_copy(shb.at[pl.ds(0, _CAP)], ohbm)
    pltpu.sync_copy(tbl_sh.at[pl.ds(wid * seg, seg)],
                    tbl_hbm.at[pl.ds(wid * seg, seg)])


@functools.lru_cache(maxsize=1)
def _get_sc_shuffle():
    mesh = plsc.VectorSubcoreMesh(
        core_axis_name="c", subcore_axis_name="s",
        num_cores=1, num_subcores=_NTILES)
    return pl.kernel(
        _shuffle_body,
        out_type=tuple(
            jax.ShapeDtypeStruct((_CAP,), jnp.float32) for _ in range(5)
        ) + (jax.ShapeDtypeStruct((_TBL,), jnp.float32),),
        mesh=mesh,
        scratch_types=[
            pltpu.VMEM((_PER_TILE,), jnp.float32),
            pltpu.VMEM((_PER_TILE,), jnp.float32),
            pltpu.VMEM((_PER_TILE,), jnp.float32),
            pltpu.VMEM((_PER_TILE,), jnp.float32),
            pltpu.VMEM((_PER_TILE,), jnp.float32),
            pltpu.VMEM((_IDXROWS, _LANES), jnp.int32),
            pltpu.VMEM((_IDXROWS, _LANES), jnp.int32),
            pltpu.VMEM((5, _IDXROWS, _LANES), jnp.int32),
            pltpu.VMEM((_CAP,), jnp.float32),
            pltpu.VMEM((_CAP,), jnp.float32),
            pltpu.VMEM_SHARED((_OUTSZ,), jnp.float32),
            pltpu.VMEM_SHARED((_OUTSZ,), jnp.float32),
            pltpu.VMEM_SHARED((_OUTSZ,), jnp.float32),
            pltpu.VMEM_SHARED((_OUTSZ,), jnp.float32),
            pltpu.VMEM_SHARED((_OUTSZ,), jnp.float32),
            pltpu.VMEM_SHARED((_TBLSZ,), jnp.float32),
            pltpu.SemaphoreType.DMA,
        ],
    )


def _fix_kernel(selr_ref, x1r_ref, y1r_ref, x2r_ref, y2r_ref, tbl_ref,
                out_ref, s2_ref, kg_ref):
    selr = selr_ref[:]
    x1r = x1r_ref[:]
    y1r = y1r_ref[:]
    x2r = x2r_ref[:]
    y2r = y2r_ref[:]
    idx_r = lax.broadcasted_iota(jnp.int32, (1, _CAP), 1)
    live_r = selr > 0.0
    a2r = jnp.maximum(x2r - x1r, 0.0) * jnp.maximum(y2r - y1r, 0.0)

    def build(bi, _):
        r0 = bi * 8
        x1c = tbl_ref[pl.ds(r0, 8), 0:1]
        y1c = tbl_ref[pl.ds(r0, 8), 1:2]
        x2c = tbl_ref[pl.ds(r0, 8), 2:3]
        y2c = tbl_ref[pl.ds(r0, 8), 3:4]
        sc = tbl_ref[pl.ds(r0, 8), 4:5]
        idx_c = r0 + lax.broadcasted_iota(jnp.int32, (8, 1), 0)
        keygt = (sc > selr) | ((sc == selr) & (idx_c < idx_r))
        kg_ref[pl.ds(r0, 8), :] = keygt.astype(jnp.float32)
        xx1 = jnp.maximum(x1c, x1r)
        yy1 = jnp.maximum(y1c, y1r)
        xx2 = jnp.minimum(x2c, x2r)
        yy2 = jnp.minimum(y2c, y2r)
        inter = jnp.maximum(xx2 - xx1, 0.0) * jnp.maximum(yy2 - yy1, 0.0)
        a1 = jnp.maximum(x2c - x1c, 0.0) * jnp.maximum(y2c - y1c, 0.0)
        union = jnp.maximum(a1 + a2r - inter, 1e-8)
        iou = inter / union
        s2 = keygt & (iou > _NMS_THRESH) & (sc > 0.0) & live_r
        s2_ref[pl.ds(r0, 8), :] = s2.astype(jnp.float32)
        return 0

    lax.fori_loop(0, _CAP // 8, build, 0)

    ones_col = jnp.ones((_CAP, 1), jnp.float32)
    dn = (((1,), (0,)), ((), ()))

    def f_cond(carry):
        _, changed = carry
        return changed > 0.0

    def f_body(carry):
        alive, _ = carry
        any_ = lax.dot_general(alive, s2_ref[:], dn,
                               preferred_element_type=jnp.float32)
        alive_new = jnp.where((any_ == 0.0) & live_r, 1.0, 0.0)
        diff = jnp.abs(alive_new - alive)
        changed = lax.dot_general(diff, ones_col, dn,
                                  preferred_element_type=jnp.float32)[0, 0]
        return (alive_new, changed)

    alive0 = jnp.where(live_r, 1.0, 0.0)
    alive, _ = lax.while_loop(f_cond, f_body, (alive0, np.float32(1.0)))

    slot = lax.dot_general(alive, kg_ref[:], dn,
                           preferred_element_type=jnp.float32)
    valid_out = (alive > 0.5) & (slot < float(_DETS))
    p_col = lax.broadcasted_iota(jnp.int32, (_OUT_ROWS, 1), 0
                                 ).astype(jnp.float32)
    pmat = jnp.where((p_col == slot) & valid_out, 1.0, 0.0)
    out_ref[:] = lax.dot_general(pmat, tbl_ref[:], dn,
                                 preferred_element_type=jnp.float32,
                                 precision=lax.Precision.HIGHEST)


@functools.partial(jax.jit, static_argnames=())
def kernel(boxes, scores):
    pad = _PAD - _N
    planes = [jnp.pad(boxes[:, c], (0, pad)) for c in range(4)]
    s2d = jnp.pad(scores, (0, pad)).reshape(_ROWS, _LANES)

    sel2d, oidx, oidxb = pl.pallas_call(
        _select_kernel,
        out_shape=(jax.ShapeDtypeStruct((_ROWS, _LANES), jnp.float32),
                   jax.ShapeDtypeStruct((_ROWS, _LANES), jnp.int32),
                   jax.ShapeDtypeStruct((_ROWS, _LANES), jnp.int32)),
    )(s2d)

    co = _get_sc_shuffle()(sel2d.reshape(_PAD), *planes,
                           oidx.reshape(_NTILES, _IDXROWS, _LANES),
                           oidxb.reshape(_NTILES, _IDXROWS, _LANES))
    rows = [a.reshape(1, _CAP) for a in co[:5]]
    tbl = co[5].reshape(_CAP, _LANES)

    out = pl.pallas_call(
        _fix_kernel,
        out_shape=jax.ShapeDtypeStruct((_OUT_ROWS, _LANES), jnp.float32),
        scratch_shapes=[
            pltpu.VMEM((_CAP, _CAP), jnp.float32),
            pltpu.VMEM((_CAP, _CAP), jnp.float32),
        ],
    )(*rows, tbl)

    out_boxes = out[:_DETS, :4]
    out_scores = out[:_DETS, 4]
    return out_boxes, out_scores


# speedup vs baseline: 2.5389x; 1.1891x over previous
import functools

import jax
import jax.numpy as jnp
import numpy as np
from jax import lax
from jax.experimental import pallas as pl
from jax.experimental.pallas import tpu as pltpu
from jax.experimental.pallas import tpu_sc as plsc

_N = 20000
_ROWS = 160
_LANES = 128
_PAD = _ROWS * _LANES
_K = 1000
_SCORE_THRESH = 0.05
_NMS_THRESH = 0.5
_DETS = 300
_OUT_ROWS = 304

_NEG = -1.0
_BIGI = 2**30
_NEGF = -3.0e38

_NTILES = 16
_PER_TILE = _PAD // _NTILES
_IDXROWS = _PER_TILE // _LANES
_CAP = 1024
_OUTSZ = _CAP + _PAD


_TBL = _CAP * _LANES
_TBLSZ = _TBL + _PAD


def _select_kernel(s_ref, sel_ref, oidx_ref, oidxb_ref):
    shape = (_ROWS, _LANES)
    row_id = lax.broadcasted_iota(jnp.int32, shape, 0)
    lane_id = lax.broadcasted_iota(jnp.int32, shape, 1)
    idx = row_id * _LANES + lane_id
    valid = idx < _N

    probs = jax.nn.sigmoid(s_ref[:])
    probs = jnp.where(probs >= _SCORE_THRESH, probs, _NEG)
    probs = jnp.where(valid, probs, _NEG)

    bits = lax.bitcast_convert_type(probs, jnp.int32)
    keys = jnp.where(probs >= 0.0, bits, np.int32(-1))
    keys = jnp.where(valid, keys, np.int32(-2))

    def bs_body(_, lh):
        lo, hi = lh
        mid = lo + (hi - lo) // 2
        c = jnp.sum(jnp.where(keys > mid, 1, 0))
        take_hi = c < _K
        return (jnp.where(take_hi, lo, mid), jnp.where(take_hi, mid, hi))

    _, t_key = lax.fori_loop(0, 31, bs_body, (np.int32(-2), np.int32(2**30)))

    c_gt = jnp.sum(jnp.where(keys > t_key, 1, 0))
    k_rem = _K - c_gt
    is_tie = keys == t_key

    def ts_body(_, lh):
        lo, hi = lh
        mid = lo + (hi - lo) // 2
        c = jnp.sum(jnp.where(is_tie & (idx < mid), 1, 0))
        take_hi = c >= k_rem
        return (jnp.where(take_hi, lo, mid), jnp.where(take_hi, mid, hi))

    _, m_hi = lax.fori_loop(0, 16, ts_body, (np.int32(0), np.int32(_PAD)))
    m_idx = jnp.where(k_rem > 0, m_hi, np.int32(0))

    cand = (keys > t_key) | (is_tie & (idx < m_idx))
    sel = jnp.where(cand, probs, _NEG)
    sel_ref[:] = sel

    live = (sel > 0.0).astype(jnp.float32)
    tri_incl = (lax.broadcasted_iota(jnp.int32, (_LANES, _LANES), 0)
                <= lax.broadcasted_iota(jnp.int32, (_LANES, _LANES), 1)
                ).astype(jnp.float32)
    within = jax.lax.dot_general(
        live, tri_incl, (((1,), (0,)), ((), ())),
        preferred_element_type=jnp.float32)
    rowtot = within[:, _LANES - 1:_LANES]
    tri_lt = (lax.broadcasted_iota(jnp.int32, (_ROWS, _ROWS), 1)
              < lax.broadcasted_iota(jnp.int32, (_ROWS, _ROWS), 0)
              ).astype(jnp.float32)
    rowpre = jax.lax.dot_general(
        tri_lt, rowtot, (((1,), (0,)), ((), ())),
        preferred_element_type=jnp.float32)
    pos = (within + rowpre - 1.0).astype(jnp.int32)
    oidx_ref[:] = jnp.where(sel > 0.0, pos, _CAP + idx)
    oidxb_ref[:] = jnp.where(sel > 0.0, pos * _LANES, _TBL + idx)


def _shuffle_body(sel_hbm, x1_hbm, y1_hbm, x2_hbm, y2_hbm, oidx_hbm,
                  oidxb_hbm,
                  osel_hbm, ox1_hbm, oy1_hbm, ox2_hbm, oy2_hbm, tbl_hbm,
                  sel_v, c1_v, c2_v, c3_v, c4_v, idx2d, idxb2d, idxbc,
                  initbuf, initz,
                  sh0, sh1, sh2, sh3, sh4, tbl_sh, sem):
    wid = lax.axis_index("s")
    base = wid * _PER_TILE

    pltpu.sync_copy(oidx_hbm.at[wid], idx2d)
    pltpu.sync_copy(oidxb_hbm.at[wid], idxb2d)
    pltpu.sync_copy(sel_hbm.at[pl.ds(base, _PER_TILE)], sel_v)
    pltpu.sync_copy(x1_hbm.at[pl.ds(base, _PER_TILE)], c1_v)
    pltpu.sync_copy(y1_hbm.at[pl.ds(base, _PER_TILE)], c2_v)
    pltpu.sync_copy(x2_hbm.at[pl.ds(base, _PER_TILE)], c3_v)
    pltpu.sync_copy(y2_hbm.at[pl.ds(base, _PER_TILE)], c4_v)

    for j in range(_IDXROWS):
        for u in range(_LANES // 16):
            v = idxb2d[j, pl.ds(u * 16, 16)]
            for c in range(5):
                idxbc[c, j, pl.ds(u * 16, 16)] = v + c

    @pl.when(wid == 0)
    def _():
        fill = jnp.full((16,), _NEG, jnp.float32)
        for u in range(_CAP // 16):
            initbuf[pl.ds(u * 16, 16)] = fill
        pltpu.sync_copy(initbuf, sh0.at[pl.ds(0, _CAP)])
        pltpu.sync_copy(initbuf, sh1.at[pl.ds(0, _CAP)])
        pltpu.sync_copy(initbuf, sh2.at[pl.ds(0, _CAP)])
        pltpu.sync_copy(initbuf, sh3.at[pl.ds(0, _CAP)])
        pltpu.sync_copy(initbuf, sh4.at[pl.ds(0, _CAP)])

    zfill = jnp.zeros((16,), jnp.float32)
    for u in range(_CAP // 16):
        initz[pl.ds(u * 16, 16)] = zfill
    seg = _TBL // _NTILES
    for k in range(seg // _CAP):
        pltpu.sync_copy(initz, tbl_sh.at[pl.ds(wid * seg + k * _CAP, _CAP)])

    plsc.subcore_barrier()

    copies = []
    for j in range(_IDXROWS):
        s = pl.ds(j * _LANES, _LANES)
        for vbuf, shb in ((sel_v, sh0), (c1_v, sh1), (c2_v, sh2),
                          (c3_v, sh3), (c4_v, sh4)):
            copies.append(
                pltpu.async_copy(vbuf.at[s], shb.at[idx2d.at[j]], sem))
        for c, vbuf in enumerate((c1_v, c2_v, c3_v, c4_v, sel_v)):
            copies.append(
                pltpu.async_copy(vbuf.at[s], tbl_sh.at[idxbc.at[c, j]], sem))
    for c in copies:
        c.wait()

    plsc.subcore_barrier()

    for a, (shb, ohbm) in enumerate(
            ((sh0, osel_hbm), (sh1, ox1_hbm), (sh2, oy1_hbm),
             (sh3, ox2_hbm), (sh4, oy2_hbm))):
        @pl.when(wid == a)
        def _(shb=shb, ohbm=ohbm):
            pltpu.sync_copy(shb.at[pl.ds(0, _CAP)], ohbm)
    pltpu.sync_copy(tbl_sh.at[pl.ds(wid * seg, seg)],
                    tbl_hbm.at[pl.ds(wid * seg, seg)])


@functools.lru_cache(maxsize=1)
def _get_sc_shuffle():
    mesh = plsc.VectorSubcoreMesh(
        core_axis_name="c", subcore_axis_name="s",
        num_cores=1, num_subcores=_NTILES)
    return pl.kernel(
        _shuffle_body,
        out_type=tuple(
            jax.ShapeDtypeStruct((_CAP,), jnp.float32) for _ in range(5)
        ) + (jax.ShapeDtypeStruct((_TBL,), jnp.float32),),
        mesh=mesh,
        scratch_types=[
            pltpu.VMEM((_PER_TILE,), jnp.float32),
            pltpu.VMEM((_PER_TILE,), jnp.float32),
            pltpu.VMEM((_PER_TILE,), jnp.float32),
            pltpu.VMEM((_PER_TILE,), jnp.float32),
            pltpu.VMEM((_PER_TILE,), jnp.float32),
            pltpu.VMEM((_IDXROWS, _LANES), jnp.int32),
            pltpu.VMEM((_IDXROWS, _LANES), jnp.int32),
            pltpu.VMEM((5, _IDXROWS, _LANES), jnp.int32),
            pltpu.VMEM((_CAP,), jnp.float32),
            pltpu.VMEM((_CAP,), jnp.float32),
            pltpu.VMEM_SHARED((_OUTSZ,), jnp.float32),
            pltpu.VMEM_SHARED((_OUTSZ,), jnp.float32),
            pltpu.VMEM_SHARED((_OUTSZ,), jnp.float32),
            pltpu.VMEM_SHARED((_OUTSZ,), jnp.float32),
            pltpu.VMEM_SHARED((_OUTSZ,), jnp.float32),
            pltpu.VMEM_SHARED((_TBLSZ,), jnp.float32),
            pltpu.SemaphoreType.DMA,
        ],
    )


def _fix_kernel(selr_ref, x1r_ref, y1r_ref, x2r_ref, y2r_ref, tbl_ref,
                out_ref, s2_ref, kg_ref):
    selr = selr_ref[:]
    x1r = x1r_ref[:]
    y1r = y1r_ref[:]
    x2r = x2r_ref[:]
    y2r = y2r_ref[:]
    idx_r = lax.broadcasted_iota(jnp.int32, (1, _CAP), 1)
    live_r = selr > 0.0
    a2r = jnp.maximum(x2r - x1r, 0.0) * jnp.maximum(y2r - y1r, 0.0)

    _BR = 16

    def build(bi, _):
        r0 = bi * _BR
        x1c = tbl_ref[pl.ds(r0, _BR), 0:1]
        y1c = tbl_ref[pl.ds(r0, _BR), 1:2]
        x2c = tbl_ref[pl.ds(r0, _BR), 2:3]
        y2c = tbl_ref[pl.ds(r0, _BR), 3:4]
        sc = tbl_ref[pl.ds(r0, _BR), 4:5]
        idx_c = r0 + lax.broadcasted_iota(jnp.int32, (_BR, 1), 0)
        keygt = (sc > selr) | ((sc == selr) & (idx_c < idx_r))
        kg_ref[pl.ds(r0, _BR), :] = keygt.astype(jnp.float32)
        xx1 = jnp.maximum(x1c, x1r)
        yy1 = jnp.maximum(y1c, y1r)
        xx2 = jnp.minimum(x2c, x2r)
        yy2 = jnp.minimum(y2c, y2r)
        inter = jnp.maximum(xx2 - xx1, 0.0) * jnp.maximum(yy2 - yy1, 0.0)
        a1 = jnp.maximum(x2c - x1c, 0.0) * jnp.maximum(y2c - y1c, 0.0)
        union = jnp.maximum(a1 + a2r - inter, 1e-8)
        iou = inter / union
        s2 = keygt & (iou > _NMS_THRESH) & (sc > 0.0) & live_r
        s2_ref[pl.ds(r0, _BR), :] = s2.astype(jnp.float32)
        return 0

    lax.fori_loop(0, _CAP // _BR, build, 0)

    ones_col = jnp.ones((_CAP, 1), jnp.float32)
    dn = (((1,), (0,)), ((), ()))

    def f_cond(carry):
        _, changed = carry
        return changed > 0.0

    def f_step(alive):
        any_ = lax.dot_general(alive, s2_ref[:], dn,
                               preferred_element_type=jnp.float32)
        return jnp.where((any_ == 0.0) & live_r, 1.0, 0.0)

    def f_body(carry):
        alive, _ = carry
        a1_ = f_step(f_step(alive))
        diff = jnp.abs(a1_ - alive)
        changed = lax.dot_general(diff, ones_col, dn,
                                  preferred_element_type=jnp.float32)[0, 0]
        return (a1_, changed)

    alive0 = jnp.where(live_r, 1.0, 0.0)
    alive, _ = lax.while_loop(f_cond, f_body, (alive0, np.float32(1.0)))

    slot = lax.dot_general(alive, kg_ref[:], dn,
                           preferred_element_type=jnp.float32)
    valid_out = (alive > 0.5) & (slot < float(_DETS))
    p_col = lax.broadcasted_iota(jnp.int32, (_OUT_ROWS, 1), 0
                                 ).astype(jnp.float32)
    pmat = jnp.where((p_col == slot) & valid_out, 1.0, 0.0)
    out_ref[:] = lax.dot_general(pmat, tbl_ref[:], dn,
                                 preferred_element_type=jnp.float32,
                                 precision=lax.Precision.HIGHEST)


@functools.partial(jax.jit, static_argnames=())
def kernel(boxes, scores):
    pad = _PAD - _N
    planes = [jnp.pad(boxes[:, c], (0, pad)) for c in range(4)]
    s2d = jnp.pad(scores, (0, pad)).reshape(_ROWS, _LANES)

    sel2d, oidx, oidxb = pl.pallas_call(
        _select_kernel,
        out_shape=(jax.ShapeDtypeStruct((_ROWS, _LANES), jnp.float32),
                   jax.ShapeDtypeStruct((_ROWS, _LANES), jnp.int32),
                   jax.ShapeDtypeStruct((_ROWS, _LANES), jnp.int32)),
    )(s2d)

    co = _get_sc_shuffle()(sel2d.reshape(_PAD), *planes,
                           oidx.reshape(_NTILES, _IDXROWS, _LANES),
                           oidxb.reshape(_NTILES, _IDXROWS, _LANES))
    rows = [a.reshape(1, _CAP) for a in co[:5]]
    tbl = co[5].reshape(_CAP, _LANES)

    out = pl.pallas_call(
        _fix_kernel,
        out_shape=jax.ShapeDtypeStruct((_OUT_ROWS, _LANES), jnp.float32),
        scratch_shapes=[
            pltpu.VMEM((_CAP, _CAP), jnp.float32),
            pltpu.VMEM((_CAP, _CAP), jnp.float32),
        ],
    )(*rows, tbl)

    out_boxes = out[:_DETS, :4]
    out_scores = out[:_DETS, 4]
    return out_boxes, out_scores
